# baseline (speedup 1.0000x reference)
import jax
import jax.numpy as jnp
from jax import lax
from jax.experimental import pallas as pl
from jax.experimental.pallas import tpu as pltpu

N_ROWS = 1024
N_COLS = 100000

_BR = 16
_NBLK = N_ROWS // _BR


def _body(x_ref, t_ref, out_ref):
    i = pl.program_id(0)
    x = x_ref[...]
    t = t_ref[...]
    col = lax.broadcasted_iota(jnp.int32, (_BR, N_COLS), 1)
    at_t = col == t
    v = jnp.max(jnp.where(at_t, x, -jnp.inf), axis=1, keepdims=True)
    contrib = (x > v) | ((x == v) & (col < t))
    rank = jnp.sum(contrib.astype(jnp.float32), axis=1, keepdims=True)
    top1 = jnp.sum((rank < 0.5).astype(jnp.float32))
    top5 = jnp.sum((rank < 4.5).astype(jnp.float32))
    part = jnp.concatenate(
        [top1.reshape(1, 1), top5.reshape(1, 1)], axis=1
    ) * (100.0 / N_ROWS)

    @pl.when(i == 0)
    def _():
        out_ref[...] = part

    @pl.when(i > 0)
    def _():
        out_ref[...] += part


@jax.jit
def kernel(pred, target):
    t2 = target.astype(jnp.int32).reshape(N_ROWS, 1)
    out = pl.pallas_call(
        _body,
        grid=(_NBLK,),
        in_specs=[
            pl.BlockSpec((_BR, N_COLS), lambda i: (i, 0)),
            pl.BlockSpec((_BR, 1), lambda i: (i, 0)),
        ],
        out_specs=pl.BlockSpec((1, 2), lambda i: (0, 0)),
        out_shape=jax.ShapeDtypeStruct((1, 2), jnp.float32),
    )(pred, t2)
    return out.reshape(2)


# speedup vs baseline: 1.0383x; 1.0383x over previous
import jax
import jax.numpy as jnp
from jax import lax
from jax.experimental import pallas as pl
from jax.experimental.pallas import tpu as pltpu

N_ROWS = 1024
N_COLS = 100000

_BR = 16
_NBLK = (N_ROWS // 2) // _BR


def _half(x, t):
    col = lax.broadcasted_iota(jnp.int32, (_BR, N_COLS), 1)
    at_t = col == t
    v = jnp.max(jnp.where(at_t, x, -jnp.inf), axis=1, keepdims=True)
    contrib = (x > v) | ((x == v) & (col < t))
    rank = jnp.sum(contrib.astype(jnp.float32), axis=1, keepdims=True)
    top1 = jnp.sum((rank < 0.5).astype(jnp.float32))
    top5 = jnp.sum((rank < 4.5).astype(jnp.float32))
    return top1, top5


def _body(xa_ref, xb_ref, ta_ref, tb_ref, out_ref):
    i = pl.program_id(0)
    t1a, t5a = _half(xa_ref[...], ta_ref[...])
    t1b, t5b = _half(xb_ref[...], tb_ref[...])
    part = jnp.concatenate(
        [(t1a + t1b).reshape(1, 1), (t5a + t5b).reshape(1, 1)], axis=1
    ) * (100.0 / N_ROWS)

    @pl.when(i == 0)
    def _():
        out_ref[...] = part

    @pl.when(i > 0)
    def _():
        out_ref[...] += part


@jax.jit
def kernel(pred, target):
    t2 = target.astype(jnp.int32).reshape(N_ROWS, 1)
    half = N_ROWS // 2
    out = pl.pallas_call(
        _body,
        grid=(_NBLK,),
        in_specs=[
            pl.BlockSpec((_BR, N_COLS), lambda i: (i, 0)),
            pl.BlockSpec((_BR, N_COLS), lambda i: (i + _NBLK, 0)),
            pl.BlockSpec((_BR, 1), lambda i: (i, 0)),
            pl.BlockSpec((_BR, 1), lambda i: (i + _NBLK, 0)),
        ],
        out_specs=pl.BlockSpec((1, 2), lambda i: (0, 0)),
        out_shape=jax.ShapeDtypeStruct((1, 2), jnp.float32),
    )(pred, pred, t2, t2)
    return out.reshape(2)
